# baseline (device time: 48551 ns/iter reference)
import functools

import jax
import jax.numpy as jnp
from jax import lax
from jax.experimental import pallas as pl
from jax.experimental.pallas import tpu as pltpu

N_DEV = 8
B, Sq, Hq, Dh = 2, 128, 4, 64
NBH = B * Hq
PACK = 128


def kernel(x, Wq, K_ext, V_ext, Wo):
    E = x.shape[-1]
    Dm = Wq.shape[-1]
    Skv_loc = K_ext.shape[1]

    def body(x_ref, wq_ref, k_ref, v_ref, wo_ref, out_ref,
             comm_ref, send_sems, recv_sems):
        my = lax.axis_index("i")
        left = lax.rem(my + N_DEV - 1, N_DEV)
        right = lax.rem(my + 1, N_DEV)

        barrier_sem = pltpu.get_barrier_semaphore()
        for nbr in (left, right):
            pl.semaphore_signal(
                barrier_sem, inc=1,
                device_id=(nbr,), device_id_type=pl.DeviceIdType.MESH,
            )
        pl.semaphore_wait(barrier_sem, 2)

        qb = lax.broadcasted_iota(jnp.int32, (Sq, Skv_loc), 0) // 64
        kb = 2 * my + lax.broadcasted_iota(jnp.int32, (Sq, Skv_loc), 1) // 64
        mask = (qb == kb) | (kb == 0) | (lax.rem(qb + kb, 3) == 0)

        o_acc, m_acc, l_acc = [], [], []
        for b in range(B):
            q_b = jnp.dot(
                x_ref[b].astype(jnp.bfloat16), wq_ref[:].astype(jnp.bfloat16),
                preferred_element_type=jnp.float32,
            )
            for h in range(Hq):
                bh = b * Hq + h
                q = q_b[:, h * Dh:(h + 1) * Dh].astype(jnp.bfloat16)
                k = k_ref[b, :, h, :].astype(jnp.bfloat16)
                v = v_ref[b, :, h, :].astype(jnp.bfloat16)
                s = lax.dot_general(
                    q, k, (((1,), (1,)), ((), ())),
                    preferred_element_type=jnp.float32,
                ) * 0.125
                s = jnp.where(mask, s, -1e9)
                m = jnp.max(s, axis=1, keepdims=True)
                w = jnp.exp(s - m)
                l = jnp.sum(w, axis=1, keepdims=True)
                o = lax.dot_general(
                    w.astype(jnp.bfloat16), v, (((1,), (0,)), ((), ())),
                    preferred_element_type=jnp.float32,
                )
                o_acc.append(o)
                m_acc.append(m)
                l_acc.append(l)
                packed = jnp.concatenate(
                    [o, m, l, jnp.zeros((Sq, PACK - Dh - 2), jnp.float32)],
                    axis=1,
                ).astype(jnp.bfloat16)
                comm_ref[0, bh] = packed

        for hop in range(N_DEV - 1):
            rdma = pltpu.make_async_remote_copy(
                src_ref=comm_ref.at[hop],
                dst_ref=comm_ref.at[hop + 1],
                send_sem=send_sems.at[hop],
                recv_sem=recv_sems.at[hop],
                device_id=(right,),
                device_id_type=pl.DeviceIdType.MESH,
            )
            rdma.start()
            rdma.wait()

            for bh in range(NBH):
                t = comm_ref[hop + 1, bh].astype(jnp.float32)
                o_in = t[:, :Dh]
                m_in = t[:, Dh:Dh + 1]
                l_in = t[:, Dh + 1:Dh + 2]
                m_new = jnp.maximum(m_acc[bh], m_in)
                sc_a = jnp.exp(m_acc[bh] - m_new)
                sc_b = jnp.exp(m_in - m_new)
                l_acc[bh] = l_acc[bh] * sc_a + l_in * sc_b
                o_acc[bh] = o_acc[bh] * sc_a + o_in * sc_b
                m_acc[bh] = m_new

        wo_bf = wo_ref[:].astype(jnp.bfloat16)
        for b in range(B):
            ctx_b = jnp.concatenate(
                [o_acc[b * Hq + h] / l_acc[b * Hq + h] for h in range(Hq)],
                axis=1,
            )
            out_ref[b] = jnp.dot(
                ctx_b.astype(jnp.bfloat16), wo_bf,
                preferred_element_type=jnp.float32,
            )

    return pl.pallas_call(
        body,
        out_shape=jax.ShapeDtypeStruct((B, Sq, E), jnp.float32),
        in_specs=[pl.BlockSpec(memory_space=pltpu.VMEM)] * 5,
        out_specs=pl.BlockSpec(memory_space=pltpu.VMEM),
        scratch_shapes=[
            pltpu.VMEM((N_DEV, NBH, Sq, PACK), jnp.bfloat16),
            pltpu.SemaphoreType.DMA((N_DEV - 1,)),
            pltpu.SemaphoreType.DMA((N_DEV - 1,)),
        ],
        compiler_params=pltpu.CompilerParams(collective_id=0),
    )(x, Wq, K_ext, V_ext, Wo)


# device time: 25290 ns/iter; 1.9198x vs baseline; 1.9198x over previous
import jax
import jax.numpy as jnp
from jax import lax
from jax.experimental import pallas as pl
from jax.experimental.pallas import tpu as pltpu

N_DEV = 8
N_ROUNDS = 3
B, Sq, Hq, Dh = 2, 128, 4, 64
NBH = B * Hq
NTILE = NBH // 2 + 1


def _pack(o, m, l):
    tiles = [
        jnp.concatenate([o[2 * p], o[2 * p + 1]], axis=1)
        for p in range(NBH // 2)
    ]
    stats = jnp.concatenate(
        [c for bh in range(NBH) for c in (m[bh], l[bh])]
        + [jnp.zeros((Sq, 128 - 2 * NBH), jnp.float32)],
        axis=1,
    )
    return jnp.stack(tiles + [stats], axis=0).astype(jnp.bfloat16)


def kernel(x, Wq, K_ext, V_ext, Wo):
    E = x.shape[-1]
    Skv_loc = K_ext.shape[1]

    def body(x_ref, wq_ref, k_ref, v_ref, wo_ref, out_ref,
             send_buf, recv_buf, send_sems, recv_sems):
        my = lax.axis_index("i")
        partners = [my ^ (1 << k) for k in range(N_ROUNDS)]

        barrier_sem = pltpu.get_barrier_semaphore()
        for p in partners:
            pl.semaphore_signal(
                barrier_sem, inc=1,
                device_id=(p,), device_id_type=pl.DeviceIdType.MESH,
            )
        pl.semaphore_wait(barrier_sem, N_ROUNDS)

        qb = lax.broadcasted_iota(jnp.int32, (Sq, Skv_loc), 0) // 64
        kb = 2 * my + lax.broadcasted_iota(jnp.int32, (Sq, Skv_loc), 1) // 64
        mask = (qb == kb) | (kb == 0) | (lax.rem(qb + kb, 3) == 0)

        o_acc, m_acc, l_acc = [], [], []
        for b in range(B):
            q_b = jnp.dot(
                x_ref[b].astype(jnp.bfloat16), wq_ref[:].astype(jnp.bfloat16),
                preferred_element_type=jnp.float32,
            )
            for h in range(Hq):
                q = q_b[:, h * Dh:(h + 1) * Dh].astype(jnp.bfloat16)
                k = k_ref[b, :, h, :].astype(jnp.bfloat16)
                v = v_ref[b, :, h, :].astype(jnp.bfloat16)
                s = lax.dot_general(
                    q, k, (((1,), (1,)), ((), ())),
                    preferred_element_type=jnp.float32,
                ) * 0.125
                s = jnp.where(mask, s, -1e9)
                m = jnp.max(s, axis=1, keepdims=True)
                w = jnp.exp(s - m)
                l = jnp.sum(w, axis=1, keepdims=True)
                o = lax.dot_general(
                    w.astype(jnp.bfloat16), v, (((1,), (0,)), ((), ())),
                    preferred_element_type=jnp.float32,
                )
                o_acc.append(o)
                m_acc.append(m)
                l_acc.append(l)

        send_buf[0] = _pack(o_acc, m_acc, l_acc)

        for r in range(N_ROUNDS):
            rdma = pltpu.make_async_remote_copy(
                src_ref=send_buf.at[r],
                dst_ref=recv_buf.at[r],
                send_sem=send_sems.at[r],
                recv_sem=recv_sems.at[r],
                device_id=(partners[r],),
                device_id_type=pl.DeviceIdType.MESH,
            )
            rdma.start()
            rdma.wait()

            stats = recv_buf[r, NBH // 2].astype(jnp.float32)
            for bh in range(NBH):
                t = recv_buf[r, bh // 2].astype(jnp.float32)
                o_in = t[:, :Dh] if bh % 2 == 0 else t[:, Dh:]
                m_in = stats[:, 2 * bh:2 * bh + 1]
                l_in = stats[:, 2 * bh + 1:2 * bh + 2]
                m_new = jnp.maximum(m_acc[bh], m_in)
                sc_a = jnp.exp(m_acc[bh] - m_new)
                sc_b = jnp.exp(m_in - m_new)
                l_acc[bh] = l_acc[bh] * sc_a + l_in * sc_b
                o_acc[bh] = o_acc[bh] * sc_a + o_in * sc_b
                m_acc[bh] = m_new
            if r + 1 < N_ROUNDS:
                send_buf[r + 1] = _pack(o_acc, m_acc, l_acc)

        wo_bf = wo_ref[:].astype(jnp.bfloat16)
        for b in range(B):
            ctx_b = jnp.concatenate(
                [o_acc[b * Hq + h] / l_acc[b * Hq + h] for h in range(Hq)],
                axis=1,
            )
            out_ref[b] = jnp.dot(
                ctx_b.astype(jnp.bfloat16), wo_bf,
                preferred_element_type=jnp.float32,
            )

    return pl.pallas_call(
        body,
        out_shape=jax.ShapeDtypeStruct((B, Sq, E), jnp.float32),
        in_specs=[pl.BlockSpec(memory_space=pltpu.VMEM)] * 5,
        out_specs=pl.BlockSpec(memory_space=pltpu.VMEM),
        scratch_shapes=[
            pltpu.VMEM((N_ROUNDS, NTILE, Sq, 128), jnp.bfloat16),
            pltpu.VMEM((N_ROUNDS, NTILE, Sq, 128), jnp.bfloat16),
            pltpu.SemaphoreType.DMA((N_ROUNDS,)),
            pltpu.SemaphoreType.DMA((N_ROUNDS,)),
        ],
        compiler_params=pltpu.CompilerParams(collective_id=0),
    )(x, Wq, K_ext, V_ext, Wo)
